# baseline (speedup 1.0000x reference)
import functools
import jax
import jax.numpy as jnp
from jax import lax
from jax.experimental import pallas as pl
from jax.experimental.pallas import tpu as pltpu
from jax.experimental.pallas import tpu_sc as plsc

_SQRT_HALF = 0.7071067811865476
_F32_EPS = float(jnp.finfo(jnp.float32).eps)


def _gelu(x):
    return 0.5 * x * (1.0 + lax.erf(x * _SQRT_HALF))


def _proj_kernel(sfeat_ref, w1b_ref, p_ref):
    p_ref[0] = lax.dot(sfeat_ref[0], w1b_ref[...],
                       preferred_element_type=jnp.float32)


def _topk_kernel(xyz_ref, sxyzt_ref, idx_ref, w_ref, *, nb, s_pts):
    b = pl.program_id(0)
    x = xyz_ref[0]
    st = sxyzt_ref[0]
    t = lax.dot(x.astype(jnp.bfloat16), st.astype(jnp.bfloat16),
                preferred_element_type=jnp.float32)
    xn = jnp.sum(x * x, axis=1, keepdims=True)
    dd = jnp.sum(st * st, axis=0, keepdims=True) - 2.0 * t

    iota = lax.broadcasted_iota(jnp.int32, dd.shape, 1)
    idxs = []
    ws = []
    wsum = jnp.zeros((nb, 1), jnp.float32)
    for _ in range(3):
        m = jnp.min(dd, axis=1, keepdims=True)
        e = dd == m
        amin = jnp.min(jnp.where(e, iota, s_pts), axis=1, keepdims=True)
        sel = iota == amin
        w = 1.0 / (jnp.maximum(m + xn, 0.0) + _F32_EPS)
        idxs.append(amin + b * s_pts)
        ws.append(w)
        wsum = wsum + w
        dd = jnp.where(sel, jnp.float32(jnp.inf), dd)
    idx_ref[0] = jnp.concatenate(idxs, axis=1)
    w_ref[0] = jnp.concatenate(ws, axis=1) / wsum


def _make_sc_gather(M, H1, QW, CH):
    NQ3 = QW * 3
    mesh = plsc.VectorSubcoreMesh(core_axis_name="c", subcore_axis_name="s")

    @functools.partial(
        pl.kernel, mesh=mesh,
        out_type=jax.ShapeDtypeStruct((M * H1,), jnp.float32),
        compiler_params=pltpu.CompilerParams(needs_layout_passes=False),
        scratch_types=[
            pltpu.VMEM((NQ3 + 16,), jnp.int32),
            pltpu.VMEM((NQ3 + 16,), jnp.float32),
            pltpu.VMEM((3 * CH, H1), jnp.float32),
            pltpu.VMEM((CH * H1,), jnp.float32),
            pltpu.SemaphoreType.DMA,
        ],
    )
    def sc_gather(idx_hbm, w_hbm, p_hbm, out_hbm, idx_v, w_v, rows_v, o_v, sem):
        nc = 2
        wid = lax.axis_index("s") * nc + lax.axis_index("c")
        qbase = wid * QW
        pltpu.sync_copy(idx_hbm.at[pl.ds(qbase * 3, NQ3)],
                        idx_v.at[pl.ds(0, NQ3)])
        pltpu.sync_copy(w_hbm.at[pl.ds(qbase * 3, NQ3)],
                        w_v.at[pl.ds(0, NQ3)])
        lanes = lax.broadcasted_iota(jnp.int32, (16,), 0)

        def chunk_body(c, carry):
            pltpu.async_copy(
                p_hbm.at[idx_v.at[pl.ds(c * 3 * CH, 3 * CH)]],
                rows_v, sem).wait()

            def q_body(q, carry2):
                woff = c * CH * 3 + q * 3
                w0 = plsc.load_gather(w_v, [jnp.full((16,), woff, jnp.int32)])
                w1 = plsc.load_gather(w_v, [jnp.full((16,), woff + 1, jnp.int32)])
                w2 = plsc.load_gather(w_v, [jnp.full((16,), woff + 2, jnp.int32)])
                r0 = q * 3
                for r in range(H1 // 16):
                    v0 = rows_v[r0, pl.ds(r * 16, 16)]
                    v1 = rows_v[r0 + 1, pl.ds(r * 16, 16)]
                    v2 = rows_v[r0 + 2, pl.ds(r * 16, 16)]
                    o_v[pl.ds(q * H1 + r * 16, 16)] = w0 * v0 + w1 * v1 + w2 * v2
                return carry2

            lax.fori_loop(0, CH, q_body, 0)
            pltpu.sync_copy(
                o_v, out_hbm.at[pl.ds((qbase + c * CH) * H1, CH * H1)])
            return carry

        lax.fori_loop(0, QW // CH, chunk_body, 0)

    return sc_gather


def _h1_kernel(pf_ref, interp_ref, w1t_ref, h1_ref, s1_ref, q1_ref, *, nb):
    i = pl.program_id(0)
    h1 = interp_ref[...] + lax.dot(pf_ref[...].astype(jnp.bfloat16),
                                   w1t_ref[...],
                                   preferred_element_type=jnp.float32)
    h1_ref[...] = h1.astype(jnp.bfloat16)

    @pl.when(i == 0)
    def _init():
        s1_ref[...] = jnp.zeros_like(s1_ref)
        q1_ref[...] = jnp.zeros_like(q1_ref)

    s1_ref[...] += jnp.sum(h1.reshape(nb // 8, 8, h1.shape[1]), axis=0)
    q1_ref[...] += jnp.sum((h1 * h1).reshape(nb // 8, 8, h1.shape[1]), axis=0)


def _bn_gelu_stats_kernel(h_ref, s_ref, q_ref, g_ref, bt_ref, w_ref,
                          s2_ref, q2_ref, *, count, nb):
    i = pl.program_id(0)
    mean = jnp.sum(s_ref[...], axis=0, keepdims=True) / count
    var = jnp.sum(q_ref[...], axis=0, keepdims=True) / count - mean * mean
    scale = g_ref[...] * lax.rsqrt(var + 1e-5)
    x = h_ref[...].astype(jnp.float32)
    xb = (x - mean) * scale + bt_ref[...]
    g = _gelu(xb).astype(jnp.bfloat16)
    h2 = lax.dot(g, w_ref[...], preferred_element_type=jnp.float32)

    @pl.when(i == 0)
    def _init():
        s2_ref[...] = jnp.zeros_like(s2_ref)
        q2_ref[...] = jnp.zeros_like(q2_ref)

    s2_ref[...] += jnp.sum(h2.reshape(nb // 8, 8, h2.shape[1]), axis=0)
    q2_ref[...] += jnp.sum((h2 * h2).reshape(nb // 8, 8, h2.shape[1]), axis=0)


def _final_kernel(h_ref, s_ref, q_ref, g_ref, bt_ref, w_ref,
                  s2_ref, q2_ref, g2_ref, bt2_ref, out_ref, *, count):
    mean = jnp.sum(s_ref[...], axis=0, keepdims=True) / count
    var = jnp.sum(q_ref[...], axis=0, keepdims=True) / count - mean * mean
    scale = g_ref[...] * lax.rsqrt(var + 1e-5)
    x = h_ref[...].astype(jnp.float32)
    xb = (x - mean) * scale + bt_ref[...]
    g = _gelu(xb).astype(jnp.bfloat16)
    h2 = lax.dot(g, w_ref[...], preferred_element_type=jnp.float32)

    mean2 = jnp.sum(s2_ref[...], axis=0, keepdims=True) / count
    var2 = jnp.sum(q2_ref[...], axis=0, keepdims=True) / count - mean2 * mean2
    scale2 = g2_ref[...] * lax.rsqrt(var2 + 1e-5)
    out_ref[...] = _gelu((h2 - mean2) * scale2 + bt2_ref[...])


def kernel(super_xyz, super_point_features, xyz, point_features,
           W1, gamma1, beta1, W2, gamma2, beta2):
    B, S, F = super_point_features.shape
    N = xyz.shape[1]
    C = point_features.shape[2]
    H1 = W1.shape[1]
    H2 = W2.shape[1]
    NB = 1024
    M = B * N
    NB2 = 2048
    NW = 32
    QW = M // NW
    CH = 64

    sxyzt = jnp.transpose(super_xyz, (0, 2, 1))
    w1_top = W1[:C].astype(jnp.bfloat16)
    w1_bot = W1[C:]
    w2_b = W2.astype(jnp.bfloat16)

    P = pl.pallas_call(
        _proj_kernel,
        grid=(B,),
        in_specs=[
            pl.BlockSpec((1, S, F), lambda b: (b, 0, 0)),
            pl.BlockSpec((F, H1), lambda b: (0, 0)),
        ],
        out_specs=pl.BlockSpec((1, S, H1), lambda b: (b, 0, 0)),
        out_shape=jax.ShapeDtypeStruct((B, S, H1), jnp.float32),
    )(super_point_features, w1_bot)

    idx, wq = pl.pallas_call(
        functools.partial(_topk_kernel, nb=NB, s_pts=S),
        grid=(B, N // NB),
        in_specs=[
            pl.BlockSpec((1, NB, 3), lambda b, n: (b, n, 0)),
            pl.BlockSpec((1, 3, S), lambda b, n: (b, 0, 0)),
        ],
        out_specs=[
            pl.BlockSpec((1, NB, 3), lambda b, n: (b, n, 0)),
            pl.BlockSpec((1, NB, 3), lambda b, n: (b, n, 0)),
        ],
        out_shape=[
            jax.ShapeDtypeStruct((B, N, 3), jnp.int32),
            jax.ShapeDtypeStruct((B, N, 3), jnp.float32),
        ],
    )(xyz, sxyzt)

    sc_gather = _make_sc_gather(M, H1, QW, CH)
    interp_flat = sc_gather(idx.reshape(M * 3), wq.reshape(M * 3),
                            P.reshape(B * S, H1))
    interp = interp_flat.reshape(M, H1)

    pf = point_features.reshape(M, C)
    h1f, s1, q1 = pl.pallas_call(
        functools.partial(_h1_kernel, nb=NB2),
        grid=(M // NB2,),
        in_specs=[
            pl.BlockSpec((NB2, C), lambda i: (i, 0)),
            pl.BlockSpec((NB2, H1), lambda i: (i, 0)),
            pl.BlockSpec((C, H1), lambda i: (0, 0)),
        ],
        out_specs=[
            pl.BlockSpec((NB2, H1), lambda i: (i, 0)),
            pl.BlockSpec((8, H1), lambda i: (0, 0)),
            pl.BlockSpec((8, H1), lambda i: (0, 0)),
        ],
        out_shape=[
            jax.ShapeDtypeStruct((M, H1), jnp.bfloat16),
            jax.ShapeDtypeStruct((8, H1), jnp.float32),
            jax.ShapeDtypeStruct((8, H1), jnp.float32),
        ],
    )(pf, interp, w1_top)

    s2, q2 = pl.pallas_call(
        functools.partial(_bn_gelu_stats_kernel, count=float(M), nb=NB2),
        grid=(M // NB2,),
        in_specs=[
            pl.BlockSpec((NB2, H1), lambda i: (i, 0)),
            pl.BlockSpec((8, H1), lambda i: (0, 0)),
            pl.BlockSpec((8, H1), lambda i: (0, 0)),
            pl.BlockSpec((1, H1), lambda i: (0, 0)),
            pl.BlockSpec((1, H1), lambda i: (0, 0)),
            pl.BlockSpec((H1, H2), lambda i: (0, 0)),
        ],
        out_specs=[
            pl.BlockSpec((8, H2), lambda i: (0, 0)),
            pl.BlockSpec((8, H2), lambda i: (0, 0)),
        ],
        out_shape=[
            jax.ShapeDtypeStruct((8, H2), jnp.float32),
            jax.ShapeDtypeStruct((8, H2), jnp.float32),
        ],
    )(h1f, s1, q1, gamma1.reshape(1, H1), beta1.reshape(1, H1), w2_b)

    out = pl.pallas_call(
        functools.partial(_final_kernel, count=float(M)),
        grid=(M // NB2,),
        in_specs=[
            pl.BlockSpec((NB2, H1), lambda i: (i, 0)),
            pl.BlockSpec((8, H1), lambda i: (0, 0)),
            pl.BlockSpec((8, H1), lambda i: (0, 0)),
            pl.BlockSpec((1, H1), lambda i: (0, 0)),
            pl.BlockSpec((1, H1), lambda i: (0, 0)),
            pl.BlockSpec((H1, H2), lambda i: (0, 0)),
            pl.BlockSpec((8, H2), lambda i: (0, 0)),
            pl.BlockSpec((8, H2), lambda i: (0, 0)),
            pl.BlockSpec((1, H2), lambda i: (0, 0)),
            pl.BlockSpec((1, H2), lambda i: (0, 0)),
        ],
        out_specs=pl.BlockSpec((NB2, H2), lambda i: (i, 0)),
        out_shape=jax.ShapeDtypeStruct((M, H2), jnp.float32),
    )(h1f, s1, q1, gamma1.reshape(1, H1), beta1.reshape(1, H1), w2_b,
      s2, q2, gamma2.reshape(1, H2), beta2.reshape(1, H2))

    return out.reshape(B, N, H2)


# speedup vs baseline: 2.5798x; 2.5798x over previous
import functools
import jax
import jax.numpy as jnp
from jax import lax
from jax.experimental import pallas as pl
from jax.experimental.pallas import tpu as pltpu

_SQRT_HALF = 0.7071067811865476
_F32_EPS = float(jnp.finfo(jnp.float32).eps)


def _gelu(x):
    return 0.5 * x * (1.0 + lax.erf(x * _SQRT_HALF))


def _topk_interp_kernel(xyz_ref, pf_ref, sxyzt_ref, sfeat_ref, w1b_ref,
                        w1t_ref, h1_ref, s1_ref, q1_ref, p_scr, *, nb):
    b = pl.program_id(0)
    n = pl.program_id(1)

    @pl.when(n == 0)
    def _proj():
        p_scr[...] = lax.dot(
            sfeat_ref[0], w1b_ref[...],
            preferred_element_type=jnp.float32).astype(jnp.bfloat16)

    x = xyz_ref[0]
    st = sxyzt_ref[0]
    t = lax.dot(x.astype(jnp.bfloat16), st.astype(jnp.bfloat16),
                preferred_element_type=jnp.float32)
    xn = jnp.sum(x * x, axis=1, keepdims=True)
    dd = jnp.sum(st * st, axis=0, keepdims=True) - 2.0 * t

    wmat = jnp.zeros_like(dd)
    wsum = jnp.zeros((nb, 1), jnp.float32)
    for _ in range(3):
        m = jnp.min(dd, axis=1, keepdims=True)
        e = dd == m
        w = 1.0 / (jnp.maximum(m + xn, 0.0) + _F32_EPS)
        wmat = jnp.where(e, jnp.broadcast_to(w, dd.shape), wmat)
        wsum = wsum + w
        dd = jnp.where(e, jnp.float32(jnp.inf), dd)
    wmat = (wmat / wsum).astype(jnp.bfloat16)

    h1 = lax.dot(wmat, p_scr[...], preferred_element_type=jnp.float32)
    h1 = h1 + lax.dot(pf_ref[0].astype(jnp.bfloat16), w1t_ref[...],
                      preferred_element_type=jnp.float32)
    h1_ref[0] = h1.astype(jnp.bfloat16)

    @pl.when((b == 0) & (n == 0))
    def _init():
        s1_ref[...] = jnp.zeros_like(s1_ref)
        q1_ref[...] = jnp.zeros_like(q1_ref)

    s1_ref[...] += jnp.sum(h1.reshape(nb // 8, 8, h1.shape[1]), axis=0)
    q1_ref[...] += jnp.sum((h1 * h1).reshape(nb // 8, 8, h1.shape[1]), axis=0)


def _bn_gelu_stats_kernel(h_ref, s_ref, q_ref, g_ref, bt_ref, w_ref,
                          s2_ref, q2_ref, *, count, nb):
    i = pl.program_id(0)
    mean = jnp.sum(s_ref[...], axis=0, keepdims=True) / count
    var = jnp.sum(q_ref[...], axis=0, keepdims=True) / count - mean * mean
    scale = g_ref[...] * lax.rsqrt(var + 1e-5)
    x = h_ref[...].astype(jnp.float32)
    xb = (x - mean) * scale + bt_ref[...]
    g = _gelu(xb).astype(jnp.bfloat16)
    h2 = lax.dot(g, w_ref[...], preferred_element_type=jnp.float32)

    @pl.when(i == 0)
    def _init():
        s2_ref[...] = jnp.zeros_like(s2_ref)
        q2_ref[...] = jnp.zeros_like(q2_ref)

    s2_ref[...] += jnp.sum(h2.reshape(nb // 8, 8, h2.shape[1]), axis=0)
    q2_ref[...] += jnp.sum((h2 * h2).reshape(nb // 8, 8, h2.shape[1]), axis=0)


def _final_kernel(h_ref, s_ref, q_ref, g_ref, bt_ref, w_ref,
                  s2_ref, q2_ref, g2_ref, bt2_ref, out_ref, *, count):
    mean = jnp.sum(s_ref[...], axis=0, keepdims=True) / count
    var = jnp.sum(q_ref[...], axis=0, keepdims=True) / count - mean * mean
    scale = g_ref[...] * lax.rsqrt(var + 1e-5)
    x = h_ref[...].astype(jnp.float32)
    xb = (x - mean) * scale + bt_ref[...]
    g = _gelu(xb).astype(jnp.bfloat16)
    h2 = lax.dot(g, w_ref[...], preferred_element_type=jnp.float32)

    mean2 = jnp.sum(s2_ref[...], axis=0, keepdims=True) / count
    var2 = jnp.sum(q2_ref[...], axis=0, keepdims=True) / count - mean2 * mean2
    scale2 = g2_ref[...] * lax.rsqrt(var2 + 1e-5)
    out_ref[...] = _gelu((h2 - mean2) * scale2 + bt2_ref[...])


def kernel(super_xyz, super_point_features, xyz, point_features,
           W1, gamma1, beta1, W2, gamma2, beta2):
    B, S, F = super_point_features.shape
    N = xyz.shape[1]
    C = point_features.shape[2]
    H1 = W1.shape[1]
    H2 = W2.shape[1]
    NB = 1024
    M = B * N
    NB2 = 4096

    sxyzt = jnp.transpose(super_xyz, (0, 2, 1))
    w1_top = W1[:C].astype(jnp.bfloat16)
    w1_bot = W1[C:]
    w2_b = W2.astype(jnp.bfloat16)

    h1, s1, q1 = pl.pallas_call(
        functools.partial(_topk_interp_kernel, nb=NB),
        grid=(B, N // NB),
        in_specs=[
            pl.BlockSpec((1, NB, 3), lambda b, n: (b, n, 0)),
            pl.BlockSpec((1, NB, C), lambda b, n: (b, n, 0)),
            pl.BlockSpec((1, 3, S), lambda b, n: (b, 0, 0)),
            pl.BlockSpec((1, S, F), lambda b, n: (b, 0, 0)),
            pl.BlockSpec((F, H1), lambda b, n: (0, 0)),
            pl.BlockSpec((C, H1), lambda b, n: (0, 0)),
        ],
        out_specs=[
            pl.BlockSpec((1, NB, H1), lambda b, n: (b, n, 0)),
            pl.BlockSpec((8, H1), lambda b, n: (0, 0)),
            pl.BlockSpec((8, H1), lambda b, n: (0, 0)),
        ],
        out_shape=[
            jax.ShapeDtypeStruct((B, N, H1), jnp.bfloat16),
            jax.ShapeDtypeStruct((8, H1), jnp.float32),
            jax.ShapeDtypeStruct((8, H1), jnp.float32),
        ],
        scratch_shapes=[pltpu.VMEM((S, H1), jnp.bfloat16)],
    )(xyz, point_features, sxyzt, super_point_features, w1_bot, w1_top)

    h1f = h1.reshape(M, H1)
    s2, q2 = pl.pallas_call(
        functools.partial(_bn_gelu_stats_kernel, count=float(M), nb=NB2),
        grid=(M // NB2,),
        in_specs=[
            pl.BlockSpec((NB2, H1), lambda i: (i, 0)),
            pl.BlockSpec((8, H1), lambda i: (0, 0)),
            pl.BlockSpec((8, H1), lambda i: (0, 0)),
            pl.BlockSpec((1, H1), lambda i: (0, 0)),
            pl.BlockSpec((1, H1), lambda i: (0, 0)),
            pl.BlockSpec((H1, H2), lambda i: (0, 0)),
        ],
        out_specs=[
            pl.BlockSpec((8, H2), lambda i: (0, 0)),
            pl.BlockSpec((8, H2), lambda i: (0, 0)),
        ],
        out_shape=[
            jax.ShapeDtypeStruct((8, H2), jnp.float32),
            jax.ShapeDtypeStruct((8, H2), jnp.float32),
        ],
    )(h1f, s1, q1, gamma1.reshape(1, H1), beta1.reshape(1, H1), w2_b)

    out = pl.pallas_call(
        functools.partial(_final_kernel, count=float(M)),
        grid=(M // NB2,),
        in_specs=[
            pl.BlockSpec((NB2, H1), lambda i: (i, 0)),
            pl.BlockSpec((8, H1), lambda i: (0, 0)),
            pl.BlockSpec((8, H1), lambda i: (0, 0)),
            pl.BlockSpec((1, H1), lambda i: (0, 0)),
            pl.BlockSpec((1, H1), lambda i: (0, 0)),
            pl.BlockSpec((H1, H2), lambda i: (0, 0)),
            pl.BlockSpec((8, H2), lambda i: (0, 0)),
            pl.BlockSpec((8, H2), lambda i: (0, 0)),
            pl.BlockSpec((1, H2), lambda i: (0, 0)),
            pl.BlockSpec((1, H2), lambda i: (0, 0)),
        ],
        out_specs=pl.BlockSpec((NB2, H2), lambda i: (i, 0)),
        out_shape=jax.ShapeDtypeStruct((M, H2), jnp.float32),
    )(h1f, s1, q1, gamma1.reshape(1, H1), beta1.reshape(1, H1), w2_b,
      s2, q2, gamma2.reshape(1, H2), beta2.reshape(1, H2))

    return out.reshape(B, N, H2)


# speedup vs baseline: 2.6065x; 1.0103x over previous
import functools
import jax
import jax.numpy as jnp
from jax import lax
from jax.experimental import pallas as pl
from jax.experimental.pallas import tpu as pltpu

_SQRT_HALF = 0.7071067811865476
_F32_EPS = float(jnp.finfo(jnp.float32).eps)


def _gelu(x):
    return 0.5 * x * (1.0 + lax.erf(x * _SQRT_HALF))


def _topk_interp_kernel(xyz_ref, pf_ref, sxyzt_ref, sfeat_ref, w1b_ref,
                        w1t_ref, h1_ref, s1_ref, q1_ref, p_scr, *, nb):
    b = pl.program_id(0)
    n = pl.program_id(1)

    @pl.when(n == 0)
    def _proj():
        p_scr[...] = lax.dot(
            sfeat_ref[0], w1b_ref[...],
            preferred_element_type=jnp.float32).astype(jnp.bfloat16)

    x = xyz_ref[0]
    st = sxyzt_ref[0]
    t = lax.dot(x.astype(jnp.bfloat16), st.astype(jnp.bfloat16),
                preferred_element_type=jnp.float32)
    xn = jnp.sum(x * x, axis=1, keepdims=True)
    dd = jnp.sum(st * st, axis=0, keepdims=True) - 2.0 * t

    wmat = jnp.zeros_like(dd)
    wsum = jnp.zeros((nb, 1), jnp.float32)
    for _ in range(3):
        m = jnp.min(dd, axis=1, keepdims=True)
        e = dd == m
        w = 1.0 / (jnp.maximum(m + xn, 0.0) + _F32_EPS)
        wmat = jnp.where(e, jnp.broadcast_to(w, dd.shape), wmat)
        wsum = wsum + w
        dd = jnp.where(e, jnp.float32(jnp.inf), dd)
    wmat = (wmat / wsum).astype(jnp.bfloat16)

    h1 = lax.dot(wmat, p_scr[...], preferred_element_type=jnp.float32)
    h1 = h1 + lax.dot(pf_ref[0].astype(jnp.bfloat16), w1t_ref[...],
                      preferred_element_type=jnp.float32)
    h1_ref[0] = h1.astype(jnp.bfloat16)

    @pl.when((b == 0) & (n == 0))
    def _init():
        s1_ref[...] = jnp.zeros_like(s1_ref)
        q1_ref[...] = jnp.zeros_like(q1_ref)

    s1_ref[...] += jnp.sum(h1.reshape(nb // 8, 8, h1.shape[1]), axis=0)
    q1_ref[...] += jnp.sum((h1 * h1).reshape(nb // 8, 8, h1.shape[1]), axis=0)


def _bn_gelu_stats_kernel(h_ref, s_ref, q_ref, g_ref, bt_ref, w_ref,
                          s2_ref, q2_ref, *, count, nb):
    i = pl.program_id(0)
    mean = jnp.sum(s_ref[...], axis=0, keepdims=True) / count
    var = jnp.sum(q_ref[...], axis=0, keepdims=True) / count - mean * mean
    scale = g_ref[...] * lax.rsqrt(var + 1e-5)
    x = h_ref[...].astype(jnp.float32)
    xb = (x - mean) * scale + bt_ref[...]
    g = _gelu(xb).astype(jnp.bfloat16)
    h2 = lax.dot(g, w_ref[...], preferred_element_type=jnp.float32)

    @pl.when(i == 0)
    def _init():
        s2_ref[...] = jnp.zeros_like(s2_ref)
        q2_ref[...] = jnp.zeros_like(q2_ref)

    s2_ref[...] += jnp.sum(h2.reshape(nb // 8, 8, h2.shape[1]), axis=0)
    q2_ref[...] += jnp.sum((h2 * h2).reshape(nb // 8, 8, h2.shape[1]), axis=0)


def _final_kernel(h_ref, s_ref, q_ref, g_ref, bt_ref, w_ref,
                  s2_ref, q2_ref, g2_ref, bt2_ref, out_ref, *, count):
    mean = jnp.sum(s_ref[...], axis=0, keepdims=True) / count
    var = jnp.sum(q_ref[...], axis=0, keepdims=True) / count - mean * mean
    scale = g_ref[...] * lax.rsqrt(var + 1e-5)
    x = h_ref[...].astype(jnp.float32)
    xb = (x - mean) * scale + bt_ref[...]
    g = _gelu(xb).astype(jnp.bfloat16)
    h2 = lax.dot(g, w_ref[...], preferred_element_type=jnp.float32)

    mean2 = jnp.sum(s2_ref[...], axis=0, keepdims=True) / count
    var2 = jnp.sum(q2_ref[...], axis=0, keepdims=True) / count - mean2 * mean2
    scale2 = g2_ref[...] * lax.rsqrt(var2 + 1e-5)
    out_ref[...] = _gelu((h2 - mean2) * scale2 + bt2_ref[...])


def kernel(super_xyz, super_point_features, xyz, point_features,
           W1, gamma1, beta1, W2, gamma2, beta2):
    B, S, F = super_point_features.shape
    N = xyz.shape[1]
    C = point_features.shape[2]
    H1 = W1.shape[1]
    H2 = W2.shape[1]
    NB = 1024
    M = B * N
    NB2 = 8192

    sxyzt = jnp.transpose(super_xyz, (0, 2, 1))
    w1_top = W1[:C].astype(jnp.bfloat16)
    w1_bot = W1[C:]
    w2_b = W2.astype(jnp.bfloat16)

    h1, s1, q1 = pl.pallas_call(
        functools.partial(_topk_interp_kernel, nb=NB),
        grid=(B, N // NB),
        in_specs=[
            pl.BlockSpec((1, NB, 3), lambda b, n: (b, n, 0)),
            pl.BlockSpec((1, NB, C), lambda b, n: (b, n, 0)),
            pl.BlockSpec((1, 3, S), lambda b, n: (b, 0, 0)),
            pl.BlockSpec((1, S, F), lambda b, n: (b, 0, 0)),
            pl.BlockSpec((F, H1), lambda b, n: (0, 0)),
            pl.BlockSpec((C, H1), lambda b, n: (0, 0)),
        ],
        out_specs=[
            pl.BlockSpec((1, NB, H1), lambda b, n: (b, n, 0)),
            pl.BlockSpec((8, H1), lambda b, n: (0, 0)),
            pl.BlockSpec((8, H1), lambda b, n: (0, 0)),
        ],
        out_shape=[
            jax.ShapeDtypeStruct((B, N, H1), jnp.bfloat16),
            jax.ShapeDtypeStruct((8, H1), jnp.float32),
            jax.ShapeDtypeStruct((8, H1), jnp.float32),
        ],
        scratch_shapes=[pltpu.VMEM((S, H1), jnp.bfloat16)],
    )(xyz, point_features, sxyzt, super_point_features, w1_bot, w1_top)

    h1f = h1.reshape(M, H1)
    s2, q2 = pl.pallas_call(
        functools.partial(_bn_gelu_stats_kernel, count=float(M), nb=NB2),
        grid=(M // NB2,),
        in_specs=[
            pl.BlockSpec((NB2, H1), lambda i: (i, 0)),
            pl.BlockSpec((8, H1), lambda i: (0, 0)),
            pl.BlockSpec((8, H1), lambda i: (0, 0)),
            pl.BlockSpec((1, H1), lambda i: (0, 0)),
            pl.BlockSpec((1, H1), lambda i: (0, 0)),
            pl.BlockSpec((H1, H2), lambda i: (0, 0)),
        ],
        out_specs=[
            pl.BlockSpec((8, H2), lambda i: (0, 0)),
            pl.BlockSpec((8, H2), lambda i: (0, 0)),
        ],
        out_shape=[
            jax.ShapeDtypeStruct((8, H2), jnp.float32),
            jax.ShapeDtypeStruct((8, H2), jnp.float32),
        ],
    )(h1f, s1, q1, gamma1.reshape(1, H1), beta1.reshape(1, H1), w2_b)

    out = pl.pallas_call(
        functools.partial(_final_kernel, count=float(M)),
        grid=(M // NB2,),
        in_specs=[
            pl.BlockSpec((NB2, H1), lambda i: (i, 0)),
            pl.BlockSpec((8, H1), lambda i: (0, 0)),
            pl.BlockSpec((8, H1), lambda i: (0, 0)),
            pl.BlockSpec((1, H1), lambda i: (0, 0)),
            pl.BlockSpec((1, H1), lambda i: (0, 0)),
            pl.BlockSpec((H1, H2), lambda i: (0, 0)),
            pl.BlockSpec((8, H2), lambda i: (0, 0)),
            pl.BlockSpec((8, H2), lambda i: (0, 0)),
            pl.BlockSpec((1, H2), lambda i: (0, 0)),
            pl.BlockSpec((1, H2), lambda i: (0, 0)),
        ],
        out_specs=pl.BlockSpec((NB2, H2), lambda i: (i, 0)),
        out_shape=jax.ShapeDtypeStruct((M, H2), jnp.float32),
    )(h1f, s1, q1, gamma1.reshape(1, H1), beta1.reshape(1, H1), w2_b,
      s2, q2, gamma2.reshape(1, H2), beta2.reshape(1, H2))

    return out.reshape(B, N, H2)


# speedup vs baseline: 2.6353x; 1.0110x over previous
import functools
import jax
import jax.numpy as jnp
from jax import lax
from jax.experimental import pallas as pl
from jax.experimental.pallas import tpu as pltpu

_SQRT_HALF = 0.7071067811865476
_F32_EPS = float(jnp.finfo(jnp.float32).eps)


def _gelu(x):
    return 0.5 * x * (1.0 + lax.erf(x * _SQRT_HALF))


def _topk_interp_kernel(xyz_ref, pf_ref, sxyzt_ref, sfeat_ref, w1b_ref,
                        w1t_ref, h1_ref, s1_ref, q1_ref, p_scr, *, nb):
    b = pl.program_id(0)
    n = pl.program_id(1)

    @pl.when(n == 0)
    def _proj():
        p_scr[...] = lax.dot(
            sfeat_ref[0], w1b_ref[...],
            preferred_element_type=jnp.float32).astype(jnp.bfloat16)

    x = xyz_ref[0]
    st = sxyzt_ref[0]
    t = lax.dot(x.astype(jnp.bfloat16), st.astype(jnp.bfloat16),
                preferred_element_type=jnp.float32)
    xn = jnp.sum(x * x, axis=1, keepdims=True)
    dd = jnp.sum(st * st, axis=0, keepdims=True) - 2.0 * t

    wmat = jnp.zeros_like(dd)
    wsum = jnp.zeros((nb, 1), jnp.float32)
    for _ in range(3):
        m = jnp.min(dd, axis=1, keepdims=True)
        e = dd == m
        w = 1.0 / (jnp.maximum(m + xn, 0.0) + _F32_EPS)
        wmat = jnp.where(e, jnp.broadcast_to(w, dd.shape), wmat)
        wsum = wsum + w
        dd = jnp.where(e, jnp.float32(jnp.inf), dd)
    wmat = (wmat / wsum).astype(jnp.bfloat16)

    h1 = lax.dot(wmat, p_scr[...], preferred_element_type=jnp.float32)
    h1 = h1 + lax.dot(pf_ref[0].astype(jnp.bfloat16), w1t_ref[...],
                      preferred_element_type=jnp.float32)
    h1_ref[0] = h1.astype(jnp.bfloat16)

    @pl.when((b == 0) & (n == 0))
    def _init():
        s1_ref[...] = jnp.zeros_like(s1_ref)
        q1_ref[...] = jnp.zeros_like(q1_ref)

    s1_ref[...] += jnp.sum(h1.reshape(nb // 8, 8, h1.shape[1]), axis=0)
    q1_ref[...] += jnp.sum((h1 * h1).reshape(nb // 8, 8, h1.shape[1]), axis=0)


def _bn_gelu_stats_kernel(h_ref, s_ref, q_ref, g_ref, bt_ref, w_ref,
                          s2_ref, q2_ref, *, count, nb):
    i = pl.program_id(0)
    mean = jnp.sum(s_ref[...], axis=0, keepdims=True) / count
    var = jnp.sum(q_ref[...], axis=0, keepdims=True) / count - mean * mean
    scale = g_ref[...] * lax.rsqrt(var + 1e-5)
    x = h_ref[...].astype(jnp.float32)
    xb = (x - mean) * scale + bt_ref[...]
    g = _gelu(xb).astype(jnp.bfloat16)
    h2 = lax.dot(g, w_ref[...], preferred_element_type=jnp.float32)

    @pl.when(i == 0)
    def _init():
        s2_ref[...] = jnp.zeros_like(s2_ref)
        q2_ref[...] = jnp.zeros_like(q2_ref)

    s2_ref[...] += jnp.sum(h2.reshape(nb // 8, 8, h2.shape[1]), axis=0)
    q2_ref[...] += jnp.sum((h2 * h2).reshape(nb // 8, 8, h2.shape[1]), axis=0)


def _final_kernel(h_ref, s_ref, q_ref, g_ref, bt_ref, w_ref,
                  s2_ref, q2_ref, g2_ref, bt2_ref, out_ref, *, count):
    mean = jnp.sum(s_ref[...], axis=0, keepdims=True) / count
    var = jnp.sum(q_ref[...], axis=0, keepdims=True) / count - mean * mean
    scale = g_ref[...] * lax.rsqrt(var + 1e-5)
    x = h_ref[...].astype(jnp.float32)
    xb = (x - mean) * scale + bt_ref[...]
    g = _gelu(xb).astype(jnp.bfloat16)
    h2 = lax.dot(g, w_ref[...], preferred_element_type=jnp.float32)

    mean2 = jnp.sum(s2_ref[...], axis=0, keepdims=True) / count
    var2 = jnp.sum(q2_ref[...], axis=0, keepdims=True) / count - mean2 * mean2
    scale2 = g2_ref[...] * lax.rsqrt(var2 + 1e-5)
    out_ref[...] = _gelu((h2 - mean2) * scale2 + bt2_ref[...])


def kernel(super_xyz, super_point_features, xyz, point_features,
           W1, gamma1, beta1, W2, gamma2, beta2):
    B, S, F = super_point_features.shape
    N = xyz.shape[1]
    C = point_features.shape[2]
    H1 = W1.shape[1]
    H2 = W2.shape[1]
    NB = 2048
    M = B * N
    NB2 = 8192

    sxyzt = jnp.transpose(super_xyz, (0, 2, 1))
    w1_top = W1[:C].astype(jnp.bfloat16)
    w1_bot = W1[C:]
    w2_b = W2.astype(jnp.bfloat16)

    h1, s1, q1 = pl.pallas_call(
        functools.partial(_topk_interp_kernel, nb=NB),
        grid=(B, N // NB),
        in_specs=[
            pl.BlockSpec((1, NB, 3), lambda b, n: (b, n, 0)),
            pl.BlockSpec((1, NB, C), lambda b, n: (b, n, 0)),
            pl.BlockSpec((1, 3, S), lambda b, n: (b, 0, 0)),
            pl.BlockSpec((1, S, F), lambda b, n: (b, 0, 0)),
            pl.BlockSpec((F, H1), lambda b, n: (0, 0)),
            pl.BlockSpec((C, H1), lambda b, n: (0, 0)),
        ],
        out_specs=[
            pl.BlockSpec((1, NB, H1), lambda b, n: (b, n, 0)),
            pl.BlockSpec((8, H1), lambda b, n: (0, 0)),
            pl.BlockSpec((8, H1), lambda b, n: (0, 0)),
        ],
        out_shape=[
            jax.ShapeDtypeStruct((B, N, H1), jnp.bfloat16),
            jax.ShapeDtypeStruct((8, H1), jnp.float32),
            jax.ShapeDtypeStruct((8, H1), jnp.float32),
        ],
        scratch_shapes=[pltpu.VMEM((S, H1), jnp.bfloat16)],
    )(xyz, point_features, sxyzt, super_point_features, w1_bot, w1_top)

    h1f = h1.reshape(M, H1)
    s2, q2 = pl.pallas_call(
        functools.partial(_bn_gelu_stats_kernel, count=float(M), nb=NB2),
        grid=(M // NB2,),
        in_specs=[
            pl.BlockSpec((NB2, H1), lambda i: (i, 0)),
            pl.BlockSpec((8, H1), lambda i: (0, 0)),
            pl.BlockSpec((8, H1), lambda i: (0, 0)),
            pl.BlockSpec((1, H1), lambda i: (0, 0)),
            pl.BlockSpec((1, H1), lambda i: (0, 0)),
            pl.BlockSpec((H1, H2), lambda i: (0, 0)),
        ],
        out_specs=[
            pl.BlockSpec((8, H2), lambda i: (0, 0)),
            pl.BlockSpec((8, H2), lambda i: (0, 0)),
        ],
        out_shape=[
            jax.ShapeDtypeStruct((8, H2), jnp.float32),
            jax.ShapeDtypeStruct((8, H2), jnp.float32),
        ],
    )(h1f, s1, q1, gamma1.reshape(1, H1), beta1.reshape(1, H1), w2_b)

    out = pl.pallas_call(
        functools.partial(_final_kernel, count=float(M)),
        grid=(M // NB2,),
        in_specs=[
            pl.BlockSpec((NB2, H1), lambda i: (i, 0)),
            pl.BlockSpec((8, H1), lambda i: (0, 0)),
            pl.BlockSpec((8, H1), lambda i: (0, 0)),
            pl.BlockSpec((1, H1), lambda i: (0, 0)),
            pl.BlockSpec((1, H1), lambda i: (0, 0)),
            pl.BlockSpec((H1, H2), lambda i: (0, 0)),
            pl.BlockSpec((8, H2), lambda i: (0, 0)),
            pl.BlockSpec((8, H2), lambda i: (0, 0)),
            pl.BlockSpec((1, H2), lambda i: (0, 0)),
            pl.BlockSpec((1, H2), lambda i: (0, 0)),
        ],
        out_specs=pl.BlockSpec((NB2, H2), lambda i: (i, 0)),
        out_shape=jax.ShapeDtypeStruct((M, H2), jnp.float32),
    )(h1f, s1, q1, gamma1.reshape(1, H1), beta1.reshape(1, H1), w2_b,
      s2, q2, gamma2.reshape(1, H2), beta2.reshape(1, H2))

    return out.reshape(B, N, H2)
